# baseline (speedup 1.0000x reference)
import functools

import jax
import jax.numpy as jnp
from jax.experimental import pallas as pl

TOP_K = 8
NUM_EXPERTS = 64
ROW_BLOCK = 256


def _router_kernel(h_ref, w_ref, b_ref, ow_ref, oi_ref):
    x = h_ref[:, :]
    w = w_ref[:, :]
    logits = jnp.dot(x, w, preferred_element_type=jnp.float32) + b_ref[0, :]

    iota = jax.lax.broadcasted_iota(jnp.int32, logits.shape, 1)
    work = logits
    vals = []
    idxs = []
    for _ in range(TOP_K):
        m = jnp.max(work, axis=-1, keepdims=True)
        idx = jnp.min(jnp.where(work == m, iota, NUM_EXPERTS), axis=-1,
                      keepdims=True)
        vals.append(m)
        idxs.append(idx)
        work = jnp.where(iota == idx, -jnp.inf, work)

    values = jnp.concatenate(vals, axis=1)
    ids = jnp.concatenate(idxs, axis=1)

    e = jnp.exp(values - values[:, :1])
    ow_ref[:, :] = e / jnp.sum(e, axis=-1, keepdims=True)
    oi_ref[:, :] = ids


@functools.partial(jax.jit, static_argnames=())
def kernel(hidden_states, weight, bias):
    n_tokens, hidden = hidden_states.shape
    grid = (n_tokens // ROW_BLOCK,)
    bias2 = bias.reshape(1, NUM_EXPERTS)
    out_w, out_i = pl.pallas_call(
        _router_kernel,
        grid=grid,
        in_specs=[
            pl.BlockSpec((ROW_BLOCK, hidden), lambda i: (i, 0)),
            pl.BlockSpec((hidden, NUM_EXPERTS), lambda i: (0, 0)),
            pl.BlockSpec((1, NUM_EXPERTS), lambda i: (0, 0)),
        ],
        out_specs=[
            pl.BlockSpec((ROW_BLOCK, TOP_K), lambda i: (i, 0)),
            pl.BlockSpec((ROW_BLOCK, TOP_K), lambda i: (i, 0)),
        ],
        out_shape=[
            jax.ShapeDtypeStruct((n_tokens, TOP_K), jnp.float32),
            jax.ShapeDtypeStruct((n_tokens, TOP_K), jnp.int32),
        ],
    )(hidden_states, weight, bias2)
    return out_w, out_i


# speedup vs baseline: 1.7952x; 1.7952x over previous
import functools

import jax
import jax.numpy as jnp
from jax.experimental import pallas as pl

TOP_K = 8
NUM_EXPERTS = 64
ROW_BLOCK = 256


def _router_kernel(h_ref, w_ref, b_ref, ow_ref, oi_ref):
    x = h_ref[:, :]
    w = w_ref[:, :]
    logits_t = jax.lax.dot_general(
        w, x, (((0,), (1,)), ((), ())),
        preferred_element_type=jnp.float32) + b_ref[:, :]

    r = logits_t.shape[1]
    eiota = jax.lax.broadcasted_iota(jnp.int32, (NUM_EXPERTS, r), 0)
    work = logits_t
    val_rows = []
    idx_rows = []
    for _ in range(TOP_K):
        m = jnp.max(work, axis=0, keepdims=True)
        idx = jnp.min(jnp.where(work == m, eiota, NUM_EXPERTS), axis=0,
                      keepdims=True)
        val_rows.append(m)
        idx_rows.append(idx)
        work = jnp.where(eiota == idx, -jnp.inf, work)

    vals = jnp.concatenate(val_rows, axis=0)
    ids = jnp.concatenate(idx_rows, axis=0)

    e = jnp.exp(vals - vals[0:1, :])
    wts = e / jnp.sum(e, axis=0, keepdims=True)

    ow_ref[:, :] = wts.T
    oi_ref[:, :] = ids.T


@functools.partial(jax.jit, static_argnames=())
def kernel(hidden_states, weight, bias):
    n_tokens, hidden = hidden_states.shape
    grid = (n_tokens // ROW_BLOCK,)
    bias2 = bias.reshape(NUM_EXPERTS, 1)
    out_w, out_i = pl.pallas_call(
        _router_kernel,
        grid=grid,
        in_specs=[
            pl.BlockSpec((ROW_BLOCK, hidden), lambda i: (i, 0)),
            pl.BlockSpec((hidden, NUM_EXPERTS), lambda i: (0, 0)),
            pl.BlockSpec((NUM_EXPERTS, 1), lambda i: (0, 0)),
        ],
        out_specs=[
            pl.BlockSpec((ROW_BLOCK, TOP_K), lambda i: (i, 0)),
            pl.BlockSpec((ROW_BLOCK, TOP_K), lambda i: (i, 0)),
        ],
        out_shape=[
            jax.ShapeDtypeStruct((n_tokens, TOP_K), jnp.float32),
            jax.ShapeDtypeStruct((n_tokens, TOP_K), jnp.int32),
        ],
    )(hidden_states, weight, bias2)
    return out_w, out_i


# speedup vs baseline: 2.2517x; 1.2543x over previous
import functools

import jax
import jax.numpy as jnp
from jax.experimental import pallas as pl

TOP_K = 8
NUM_EXPERTS = 64
ROW_BLOCK = 512


def _router_kernel(h_ref, w_ref, b_ref, ow_ref, oi_ref):
    x = h_ref[:, :]
    w = w_ref[:, :]
    logits_t = jax.lax.dot_general(
        w, x, (((0,), (1,)), ((), ())),
        preferred_element_type=jnp.float32) + b_ref[:, :]

    r = logits_t.shape[1]
    eiota = jax.lax.broadcasted_iota(jnp.int32, (NUM_EXPERTS, r), 0)
    work = logits_t
    val_rows = []
    idx_rows = []
    for _ in range(TOP_K):
        m = jnp.max(work, axis=0, keepdims=True)
        idx = jnp.min(jnp.where(work == m, eiota, NUM_EXPERTS), axis=0,
                      keepdims=True)
        val_rows.append(m)
        idx_rows.append(idx)
        work = jnp.where(eiota == idx, -jnp.inf, work)

    vals = jnp.concatenate(val_rows, axis=0)
    ids = jnp.concatenate(idx_rows, axis=0)

    e = jnp.exp(vals - vals[0:1, :])
    wts = e / jnp.sum(e, axis=0, keepdims=True)

    ow_ref[:, :] = wts.T
    oi_ref[:, :] = ids.T


@functools.partial(jax.jit, static_argnames=())
def kernel(hidden_states, weight, bias):
    n_tokens, hidden = hidden_states.shape
    grid = (n_tokens // ROW_BLOCK,)
    bias2 = bias.reshape(NUM_EXPERTS, 1)
    out_w, out_i = pl.pallas_call(
        _router_kernel,
        grid=grid,
        in_specs=[
            pl.BlockSpec((ROW_BLOCK, hidden), lambda i: (i, 0)),
            pl.BlockSpec((hidden, NUM_EXPERTS), lambda i: (0, 0)),
            pl.BlockSpec((NUM_EXPERTS, 1), lambda i: (0, 0)),
        ],
        out_specs=[
            pl.BlockSpec((ROW_BLOCK, TOP_K), lambda i: (i, 0)),
            pl.BlockSpec((ROW_BLOCK, TOP_K), lambda i: (i, 0)),
        ],
        out_shape=[
            jax.ShapeDtypeStruct((n_tokens, TOP_K), jnp.float32),
            jax.ShapeDtypeStruct((n_tokens, TOP_K), jnp.int32),
        ],
    )(hidden_states, weight, bias2)
    return out_w, out_i


# speedup vs baseline: 2.5364x; 1.1264x over previous
import functools

import jax
import jax.numpy as jnp
from jax.experimental import pallas as pl

TOP_K = 8
NUM_EXPERTS = 64
ROW_BLOCK = 1024


def _router_kernel(h_ref, w_ref, b_ref, ow_ref, oi_ref):
    x = h_ref[:, :]
    w = w_ref[:, :]
    logits_t = jax.lax.dot_general(
        w, x, (((0,), (1,)), ((), ())),
        preferred_element_type=jnp.float32) + b_ref[:, :]

    r = logits_t.shape[1]
    eiota = jax.lax.broadcasted_iota(jnp.int32, (NUM_EXPERTS, r), 0)
    work = logits_t
    val_rows = []
    idx_rows = []
    for _ in range(TOP_K):
        m = jnp.max(work, axis=0, keepdims=True)
        idx = jnp.min(jnp.where(work == m, eiota, NUM_EXPERTS), axis=0,
                      keepdims=True)
        val_rows.append(m)
        idx_rows.append(idx)
        work = jnp.where(eiota == idx, -jnp.inf, work)

    vals = jnp.concatenate(val_rows, axis=0)
    ids = jnp.concatenate(idx_rows, axis=0)

    e = jnp.exp(vals - vals[0:1, :])
    wts = e / jnp.sum(e, axis=0, keepdims=True)

    ow_ref[:, :] = wts.T
    oi_ref[:, :] = ids.T


@functools.partial(jax.jit, static_argnames=())
def kernel(hidden_states, weight, bias):
    n_tokens, hidden = hidden_states.shape
    grid = (n_tokens // ROW_BLOCK,)
    bias2 = bias.reshape(NUM_EXPERTS, 1)
    out_w, out_i = pl.pallas_call(
        _router_kernel,
        grid=grid,
        in_specs=[
            pl.BlockSpec((ROW_BLOCK, hidden), lambda i: (i, 0)),
            pl.BlockSpec((hidden, NUM_EXPERTS), lambda i: (0, 0)),
            pl.BlockSpec((NUM_EXPERTS, 1), lambda i: (0, 0)),
        ],
        out_specs=[
            pl.BlockSpec((ROW_BLOCK, TOP_K), lambda i: (i, 0)),
            pl.BlockSpec((ROW_BLOCK, TOP_K), lambda i: (i, 0)),
        ],
        out_shape=[
            jax.ShapeDtypeStruct((n_tokens, TOP_K), jnp.float32),
            jax.ShapeDtypeStruct((n_tokens, TOP_K), jnp.int32),
        ],
    )(hidden_states, weight, bias2)
    return out_w, out_i


# speedup vs baseline: 2.5717x; 1.0139x over previous
import functools

import jax
import jax.numpy as jnp
from jax.experimental import pallas as pl

TOP_K = 8
NUM_EXPERTS = 64
ROW_BLOCK = 2048


def _router_kernel(h_ref, w_ref, b_ref, ow_ref, oi_ref):
    x = h_ref[:, :]
    w = w_ref[:, :]
    logits_t = jax.lax.dot_general(
        w, x, (((0,), (1,)), ((), ())),
        preferred_element_type=jnp.float32) + b_ref[:, :]

    r = logits_t.shape[1]
    eiota = jax.lax.broadcasted_iota(jnp.int32, (NUM_EXPERTS, r), 0)
    work = logits_t
    val_rows = []
    idx_rows = []
    for _ in range(TOP_K):
        m = jnp.max(work, axis=0, keepdims=True)
        idx = jnp.min(jnp.where(work == m, eiota, NUM_EXPERTS), axis=0,
                      keepdims=True)
        val_rows.append(m)
        idx_rows.append(idx)
        work = jnp.where(eiota == idx, -jnp.inf, work)

    vals = jnp.concatenate(val_rows, axis=0)
    ids = jnp.concatenate(idx_rows, axis=0)

    e = jnp.exp(vals - vals[0:1, :])
    wts = e / jnp.sum(e, axis=0, keepdims=True)

    ow_ref[:, :] = wts.T
    oi_ref[:, :] = ids.T


@functools.partial(jax.jit, static_argnames=())
def kernel(hidden_states, weight, bias):
    n_tokens, hidden = hidden_states.shape
    grid = (n_tokens // ROW_BLOCK,)
    bias2 = bias.reshape(NUM_EXPERTS, 1)
    out_w, out_i = pl.pallas_call(
        _router_kernel,
        grid=grid,
        in_specs=[
            pl.BlockSpec((ROW_BLOCK, hidden), lambda i: (i, 0)),
            pl.BlockSpec((hidden, NUM_EXPERTS), lambda i: (0, 0)),
            pl.BlockSpec((NUM_EXPERTS, 1), lambda i: (0, 0)),
        ],
        out_specs=[
            pl.BlockSpec((ROW_BLOCK, TOP_K), lambda i: (i, 0)),
            pl.BlockSpec((ROW_BLOCK, TOP_K), lambda i: (i, 0)),
        ],
        out_shape=[
            jax.ShapeDtypeStruct((n_tokens, TOP_K), jnp.float32),
            jax.ShapeDtypeStruct((n_tokens, TOP_K), jnp.int32),
        ],
    )(hidden_states, weight, bias2)
    return out_w, out_i
